# baseline (speedup 1.0000x reference)
import functools

import jax
import jax.numpy as jnp
from jax import lax
from jax.experimental import pallas as pl
from jax.experimental.pallas import tpu as pltpu
from jax.experimental.pallas import tpu_sc as plsc

N = 10000
NC = 2
NS = 16
ZR = N // NS
E = 320000
CH = 128
ER = E // CH
CPT = ER // NS
KB = 52
NB = CPT // KB
DW = 16
HW = 64
R = 2000


def _edge_loop(src_hbm, dst_hbm, xtab, acc, sidx, didx,
               rows, gsems, ssems, s, deg=None):
    base = s * CPT
    rn = len(rows)

    def one_block(bb):
        off = pl.ds(base + bb * KB, KB)
        pltpu.sync_copy(src_hbm.at[off], sidx)
        pltpu.sync_copy(dst_hbm.at[off], didx)
        for ch in range(rn - 1):
            pltpu.async_copy(xtab.at[sidx.at[ch]], rows[ch], gsems[ch])
        for ch in range(KB):
            j = ch % rn
            pltpu.make_async_copy(xtab.at[sidx.at[ch]], rows[j], gsems[j]).wait()
            pltpu.async_copy(rows[j], acc.at[didx.at[ch]], ssems[j], add=True)
            if deg is not None:
                c, orows, dacc = deg

                @pl.when((ch < KB // 2) == (c == 0))
                def _():
                    pltpu.sync_copy(orows, dacc.at[didx.at[ch]], add=True)
            if ch + rn - 1 < KB:
                k = (ch + rn - 1) % rn
                if ch >= 1:
                    pltpu.make_async_copy(rows[k], acc.at[didx.at[ch - 1]],
                                          ssems[k]).wait()
                pltpu.async_copy(xtab.at[sidx.at[ch + rn - 1]], rows[k], gsems[k])
        for ch in range(KB - rn, KB):
            pltpu.make_async_copy(rows[ch % rn], acc.at[didx.at[ch]],
                                  ssems[ch % rn]).wait()

    for bb in range(NB):
        one_block(bb)

    @pl.when(s < ER - NS * CPT)
    def _():
        pltpu.sync_copy(src_hbm.at[pl.ds(NS * CPT + s, 1)], sidx.at[pl.ds(0, 1)])
        pltpu.sync_copy(dst_hbm.at[pl.ds(NS * CPT + s, 1)], didx.at[pl.ds(0, 1)])
        pltpu.async_copy(xtab.at[sidx.at[0]], rows[0], gsems[0]).wait()
        pltpu.sync_copy(rows[0], acc.at[didx.at[0]], add=True)
        if deg is not None:
            c, orows, dacc = deg

            @pl.when(c == 0)
            def _():
                pltpu.sync_copy(orows, dacc.at[didx.at[0]], add=True)


def _sc_agg1_body(x_hbm, ones_hbm, src_hbm, dst_hbm, z_hbm, out0_hbm, out1_hbm,
                  xtab, acc, dacc, orows, sidx, didx,
                  rows0, rows1, rows2, g0, g1, g2, s0, s1, s2):
    c = lax.axis_index("c")
    s = lax.axis_index("s")
    rs = pl.ds(s * ZR, ZR)

    pltpu.sync_copy(x_hbm.at[rs, pl.ds(c * HW, HW)], xtab.at[rs])
    pltpu.sync_copy(z_hbm.at[:, pl.ds(0, HW)], acc.at[rs])
    pltpu.sync_copy(z_hbm.at[:, pl.ds(0, DW)], dacc.at[rs])
    pltpu.sync_copy(ones_hbm.at[pl.ds(0, CH), pl.ds(0, DW)], orows)
    plsc.subcore_barrier()

    _edge_loop(src_hbm, dst_hbm, xtab, acc, sidx, didx,
               (rows0, rows1, rows2), (g0, g1, g2), (s0, s1, s2), s,
               deg=(c, orows, dacc))

    plsc.subcore_barrier()

    @pl.when(c == 0)
    def _():
        pltpu.sync_copy(acc.at[rs], out0_hbm.at[rs, pl.ds(0, HW)])
        pltpu.sync_copy(dacc.at[rs], out0_hbm.at[rs, pl.ds(HW, DW)])

    @pl.when(c == 1)
    def _():
        pltpu.sync_copy(acc.at[rs], out1_hbm.at[rs, pl.ds(0, HW)])
        pltpu.sync_copy(dacc.at[rs], out1_hbm.at[rs, pl.ds(HW, DW)])


def _sc_agg2_body(x_hbm, src_hbm, dst_hbm, z_hbm, out_hbm,
                  xtab, acc, sidx, didx,
                  rows0, rows1, rows2, rows3, g0, g1, g2, g3, s0, s1, s2, s3):
    c = lax.axis_index("c")
    s = lax.axis_index("s")
    rs = pl.ds(s * ZR, ZR)
    cs = pl.ds(c * HW, HW)

    pltpu.sync_copy(x_hbm.at[rs, cs], xtab.at[rs])
    pltpu.sync_copy(z_hbm.at[:, pl.ds(0, HW)], acc.at[rs])
    plsc.subcore_barrier()

    _edge_loop(src_hbm, dst_hbm, xtab, acc, sidx, didx,
               (rows0, rows1, rows2, rows3), (g0, g1, g2, g3),
               (s0, s1, s2, s3), s)

    plsc.subcore_barrier()
    pltpu.sync_copy(acc.at[rs], out_hbm.at[rs, cs])


_SC_MESH = plsc.VectorSubcoreMesh(core_axis_name="c", subcore_axis_name="s")
_SC_PARAMS = pltpu.CompilerParams(use_tc_tiling_on_sc=False)


def _rows_sems_scratch(rn):
    return ([pltpu.VMEM((CH, HW), jnp.float32)] * rn
            + [pltpu.SemaphoreType.DMA] * (2 * rn))


_agg1 = pl.kernel(
    _sc_agg1_body,
    out_type=[jax.ShapeDtypeStruct((N, 128), jnp.float32),
              jax.ShapeDtypeStruct((N, 128), jnp.float32)],
    mesh=_SC_MESH,
    scratch_types=[
        pltpu.VMEM_SHARED((N, HW), jnp.float32),
        pltpu.VMEM_SHARED((N, HW), jnp.float32),
        pltpu.VMEM_SHARED((N, DW), jnp.float32),
        pltpu.VMEM((CH, DW), jnp.float32),
        pltpu.VMEM((KB, CH), jnp.int32),
        pltpu.VMEM((KB, CH), jnp.int32),
    ] + _rows_sems_scratch(3),
    compiler_params=_SC_PARAMS,
    name="sage_sc_agg1",
)

_agg2 = pl.kernel(
    _sc_agg2_body,
    out_type=jax.ShapeDtypeStruct((N, 128), jnp.float32),
    mesh=_SC_MESH,
    scratch_types=[
        pltpu.VMEM_SHARED((N, HW), jnp.float32),
        pltpu.VMEM_SHARED((N, HW), jnp.float32),
        pltpu.VMEM((KB, CH), jnp.int32),
        pltpu.VMEM((KB, CH), jnp.int32),
    ] + _rows_sems_scratch(4),
    compiler_params=_SC_PARAMS,
    name="sage_sc_agg2",
)


def _tc1_body(sp0_ref, sp1_ref, x_ref, wa_ref, wb_ref, b_ref, o_ref, invd_ref):
    v0 = sp0_ref[...]
    v1 = sp1_ref[...]
    ssum = jnp.concatenate([v0[:, :HW], v1[:, :HW]], axis=1)
    deg = v0[:, HW:HW + 1] + v1[:, HW:HW + 1]
    invd = 1.0 / jnp.maximum(deg, 1.0)
    mean = ssum * invd
    y = (jnp.dot(x_ref[...], wa_ref[...], preferred_element_type=jnp.float32)
         + jnp.dot(mean, wb_ref[...], preferred_element_type=jnp.float32)
         + b_ref[...])
    o_ref[...] = jnp.maximum(y, 0.0)
    invd_ref[...] = jnp.broadcast_to(invd, (invd.shape[0], 128))


def _tc23_body(relu, sp_ref, x_ref, invd_ref, wa_ref, wb_ref, b_ref, o_ref):
    mean = sp_ref[...] * invd_ref[...]
    y = (jnp.dot(x_ref[...], wa_ref[...], preferred_element_type=jnp.float32)
         + jnp.dot(mean, wb_ref[...], preferred_element_type=jnp.float32)
         + b_ref[...])
    if relu:
        y = jnp.maximum(y, 0.0)
    o_ref[...] = y


_MAT_SPEC = pl.BlockSpec((128, 128), lambda i: (0, 0))
_VEC_SPEC = pl.BlockSpec((1, 128), lambda i: (0, 0))
_ROW_SPEC = pl.BlockSpec((R, 128), lambda i: (i, 0))


def _tc1(sp0, sp1, x, wa, wb, b):
    return pl.pallas_call(
        _tc1_body,
        grid=(N // R,),
        in_specs=[_ROW_SPEC, _ROW_SPEC, _ROW_SPEC, _MAT_SPEC, _MAT_SPEC, _VEC_SPEC],
        out_specs=[_ROW_SPEC, _ROW_SPEC],
        out_shape=[
            jax.ShapeDtypeStruct((N, 128), jnp.float32),
            jax.ShapeDtypeStruct((N, 128), jnp.float32),
        ],
        name="sage_tc1",
    )(sp0, sp1, x, wa, wb, b)


def _tc23(sp, x, invd, wa, wb, b, relu):
    return pl.pallas_call(
        functools.partial(_tc23_body, relu),
        grid=(N // R,),
        in_specs=[_ROW_SPEC, _ROW_SPEC, _ROW_SPEC, _MAT_SPEC, _MAT_SPEC, _VEC_SPEC],
        out_specs=_ROW_SPEC,
        out_shape=jax.ShapeDtypeStruct((N, 128), jnp.float32),
        name="sage_tc23",
    )(sp, x, invd, wa, wb, b)


def kernel(h, edge_index, W1, b1, W2, b2, W3, b3):
    f32 = jnp.float32
    src_r = edge_index[0].reshape(ER, CH)
    dst_r = edge_index[1].reshape(ER, CH)

    ones = jnp.ones((ZR, 128), f32)
    z = jnp.zeros((ZR, 128), f32)

    sp0, sp1 = _agg1(h, ones, src_r, dst_r, z)
    x1, invd = _tc1(sp0, sp1, h, W1[:128], W1[128:], b1.reshape(1, 128))
    s2 = _agg2(x1, src_r, dst_r, z)
    x2 = _tc23(s2, x1, invd, W2[:128], W2[128:], b2.reshape(1, 128), True)
    s3 = _agg2(x2, src_r, dst_r, z)
    x3 = _tc23(s3, x2, invd, W3[:128], W3[128:], b3.reshape(1, 128), False)
    return x3


# speedup vs baseline: 1.0146x; 1.0146x over previous
import functools

import jax
import jax.numpy as jnp
from jax import lax
from jax.experimental import pallas as pl
from jax.experimental.pallas import tpu as pltpu
from jax.experimental.pallas import tpu_sc as plsc

N = 10000
NC = 2
NS = 16
ZR = N // NS
E = 320000
CH = 128
ER = E // CH
CPT = ER // NS
KB = 39
NB = CPT // KB
DW = 16
HW = 64
R = 2000


def _edge_loop(src_hbm, dst_hbm, xtab, acc, sidxs, didxs,
               rows, gsems, ssems, semi, s, deg=None, idx_dbuf=True):
    base = s * CPT
    npre = 2 if idx_dbuf else 1

    for p in range(npre):
        pltpu.async_copy(src_hbm.at[pl.ds(base + p * KB, KB)], sidxs[p], semi)
        pltpu.async_copy(dst_hbm.at[pl.ds(base + p * KB, KB)], didxs[p], semi)

    def one_block(bb, p):
        off = pl.ds(base + bb * KB, KB)
        pltpu.make_async_copy(src_hbm.at[off], sidxs[p], semi).wait()
        pltpu.make_async_copy(dst_hbm.at[off], didxs[p], semi).wait()
        sidx = sidxs[p]
        didx = didxs[p]
        pltpu.async_copy(xtab.at[sidx.at[0]], rows[0], gsems[0])
        pltpu.async_copy(xtab.at[sidx.at[1]], rows[1], gsems[1])
        for ch in range(KB):
            j = ch % 3
            pltpu.make_async_copy(xtab.at[sidx.at[ch]], rows[j], gsems[j]).wait()
            pltpu.async_copy(rows[j], acc.at[didx.at[ch]], ssems[j], add=True)
            if deg is not None:
                c, orows, dacc, semd = deg

                @pl.when((bb < NB // 2) == (c == 0))
                def _():
                    if ch > 0:
                        pltpu.make_async_copy(
                            orows, dacc.at[didx.at[ch - 1]], semd).wait()
                    pltpu.async_copy(orows, dacc.at[didx.at[ch]], semd, add=True)
            if ch + 2 < KB:
                k = (ch + 2) % 3
                if ch >= 1:
                    pltpu.make_async_copy(rows[k], acc.at[didx.at[ch - 1]],
                                          ssems[k]).wait()
                pltpu.async_copy(xtab.at[sidx.at[ch + 2]], rows[k], gsems[k])
        for ch in (KB - 3, KB - 2, KB - 1):
            pltpu.make_async_copy(rows[ch % 3], acc.at[didx.at[ch]],
                                  ssems[ch % 3]).wait()
        if deg is not None:
            c, orows, dacc, semd = deg

            @pl.when((bb < NB // 2) == (c == 0))
            def _():
                pltpu.make_async_copy(orows, dacc.at[didx.at[KB - 1]],
                                      semd).wait()

    if idx_dbuf:
        def blk2(b2, carry):
            for p in range(2):
                bb = b2 * 2 + p
                one_block(bb, p)

                @pl.when(bb + 2 < NB)
                def _():
                    off2 = pl.ds(base + (bb + 2) * KB, KB)
                    pltpu.async_copy(src_hbm.at[off2], sidxs[p], semi)
                    pltpu.async_copy(dst_hbm.at[off2], didxs[p], semi)
            return carry

        lax.fori_loop(0, NB // 2, blk2, 0)
    else:
        def blk1(b, carry):
            one_block(b, 0)

            @pl.when(b + 1 < NB)
            def _():
                off2 = pl.ds(base + (b + 1) * KB, KB)
                pltpu.async_copy(src_hbm.at[off2], sidxs[0], semi)
                pltpu.async_copy(dst_hbm.at[off2], didxs[0], semi)
            return carry

        lax.fori_loop(0, NB, blk1, 0)

    @pl.when(s < ER - NS * CPT)
    def _():
        pltpu.sync_copy(src_hbm.at[pl.ds(NS * CPT + s, 1)], sidxs[0].at[pl.ds(0, 1)])
        pltpu.sync_copy(dst_hbm.at[pl.ds(NS * CPT + s, 1)], didxs[0].at[pl.ds(0, 1)])
        pltpu.async_copy(xtab.at[sidxs[0].at[0]], rows[0], gsems[0]).wait()
        pltpu.sync_copy(rows[0], acc.at[didxs[0].at[0]], add=True)
        if deg is not None:
            c, orows, dacc, semd = deg

            @pl.when(c == 0)
            def _():
                pltpu.sync_copy(orows, dacc.at[didxs[0].at[0]], add=True)


def _sc_agg1_body(x_hbm, ones_hbm, src_hbm, dst_hbm, z_hbm, out0_hbm, out1_hbm,
                  xtab, acc, dacc, orows, sidx0, didx0,
                  rows0, rows1, rows2, g0, g1, g2, s0, s1, s2, semi, semd):
    c = lax.axis_index("c")
    s = lax.axis_index("s")
    rs = pl.ds(s * ZR, ZR)

    pltpu.sync_copy(x_hbm.at[rs, pl.ds(c * HW, HW)], xtab.at[rs])
    pltpu.sync_copy(z_hbm.at[:, pl.ds(0, HW)], acc.at[rs])
    pltpu.sync_copy(z_hbm.at[:, pl.ds(0, DW)], dacc.at[rs])
    pltpu.sync_copy(ones_hbm.at[pl.ds(0, CH), pl.ds(0, DW)], orows)
    plsc.subcore_barrier()

    _edge_loop(src_hbm, dst_hbm, xtab, acc, (sidx0,), (didx0,),
               (rows0, rows1, rows2), (g0, g1, g2), (s0, s1, s2), semi, s,
               deg=(c, orows, dacc, semd), idx_dbuf=False)

    plsc.subcore_barrier()

    @pl.when(c == 0)
    def _():
        pltpu.sync_copy(acc.at[rs], out0_hbm.at[rs, pl.ds(0, HW)])
        pltpu.sync_copy(dacc.at[rs], out0_hbm.at[rs, pl.ds(HW, DW)])

    @pl.when(c == 1)
    def _():
        pltpu.sync_copy(acc.at[rs], out1_hbm.at[rs, pl.ds(0, HW)])
        pltpu.sync_copy(dacc.at[rs], out1_hbm.at[rs, pl.ds(HW, DW)])


def _sc_agg2_body(x_hbm, src_hbm, dst_hbm, z_hbm, out_hbm,
                  xtab, acc, sidx0, didx0, sidx1, didx1,
                  rows0, rows1, rows2, g0, g1, g2, s0, s1, s2, semi):
    c = lax.axis_index("c")
    s = lax.axis_index("s")
    rs = pl.ds(s * ZR, ZR)
    cs = pl.ds(c * HW, HW)

    pltpu.sync_copy(x_hbm.at[rs, cs], xtab.at[rs])
    pltpu.sync_copy(z_hbm.at[:, pl.ds(0, HW)], acc.at[rs])
    plsc.subcore_barrier()

    _edge_loop(src_hbm, dst_hbm, xtab, acc, (sidx0, sidx1), (didx0, didx1),
               (rows0, rows1, rows2), (g0, g1, g2), (s0, s1, s2), semi, s)

    plsc.subcore_barrier()
    pltpu.sync_copy(acc.at[rs], out_hbm.at[rs, cs])


_SC_MESH = plsc.VectorSubcoreMesh(core_axis_name="c", subcore_axis_name="s")
_SC_PARAMS = pltpu.CompilerParams(use_tc_tiling_on_sc=False)


def _rows_sems_scratch():
    return [
        pltpu.VMEM((CH, HW), jnp.float32),
        pltpu.VMEM((CH, HW), jnp.float32),
        pltpu.VMEM((CH, HW), jnp.float32),
        pltpu.SemaphoreType.DMA,
        pltpu.SemaphoreType.DMA,
        pltpu.SemaphoreType.DMA,
        pltpu.SemaphoreType.DMA,
        pltpu.SemaphoreType.DMA,
        pltpu.SemaphoreType.DMA,
        pltpu.SemaphoreType.DMA,
    ]


_agg1 = pl.kernel(
    _sc_agg1_body,
    out_type=[jax.ShapeDtypeStruct((N, 128), jnp.float32),
              jax.ShapeDtypeStruct((N, 128), jnp.float32)],
    mesh=_SC_MESH,
    scratch_types=[
        pltpu.VMEM_SHARED((N, HW), jnp.float32),
        pltpu.VMEM_SHARED((N, HW), jnp.float32),
        pltpu.VMEM_SHARED((N, DW), jnp.float32),
        pltpu.VMEM((CH, DW), jnp.float32),
        pltpu.VMEM((KB, CH), jnp.int32),
        pltpu.VMEM((KB, CH), jnp.int32),
    ] + _rows_sems_scratch() + [pltpu.SemaphoreType.DMA],
    compiler_params=_SC_PARAMS,
    name="sage_sc_agg1",
)

_agg2 = pl.kernel(
    _sc_agg2_body,
    out_type=jax.ShapeDtypeStruct((N, 128), jnp.float32),
    mesh=_SC_MESH,
    scratch_types=[
        pltpu.VMEM_SHARED((N, HW), jnp.float32),
        pltpu.VMEM_SHARED((N, HW), jnp.float32),
        pltpu.VMEM((KB, CH), jnp.int32),
        pltpu.VMEM((KB, CH), jnp.int32),
        pltpu.VMEM((KB, CH), jnp.int32),
        pltpu.VMEM((KB, CH), jnp.int32),
    ] + _rows_sems_scratch(),
    compiler_params=_SC_PARAMS,
    name="sage_sc_agg2",
)


def _tc1_body(sp0_ref, sp1_ref, x_ref, wa_ref, wb_ref, b_ref, o_ref, invd_ref):
    v0 = sp0_ref[...]
    v1 = sp1_ref[...]
    ssum = jnp.concatenate([v0[:, :HW], v1[:, :HW]], axis=1)
    deg = v0[:, HW:HW + 1] + v1[:, HW:HW + 1]
    invd = 1.0 / jnp.maximum(deg, 1.0)
    mean = ssum * invd
    y = (jnp.dot(x_ref[...], wa_ref[...], preferred_element_type=jnp.float32)
         + jnp.dot(mean, wb_ref[...], preferred_element_type=jnp.float32)
         + b_ref[...])
    o_ref[...] = jnp.maximum(y, 0.0)
    invd_ref[...] = jnp.broadcast_to(invd, (invd.shape[0], 128))


def _tc23_body(relu, sp_ref, x_ref, invd_ref, wa_ref, wb_ref, b_ref, o_ref):
    mean = sp_ref[...] * invd_ref[...]
    y = (jnp.dot(x_ref[...], wa_ref[...], preferred_element_type=jnp.float32)
         + jnp.dot(mean, wb_ref[...], preferred_element_type=jnp.float32)
         + b_ref[...])
    if relu:
        y = jnp.maximum(y, 0.0)
    o_ref[...] = y


_MAT_SPEC = pl.BlockSpec((128, 128), lambda i: (0, 0))
_VEC_SPEC = pl.BlockSpec((1, 128), lambda i: (0, 0))
_ROW_SPEC = pl.BlockSpec((R, 128), lambda i: (i, 0))


def _tc1(sp0, sp1, x, wa, wb, b):
    return pl.pallas_call(
        _tc1_body,
        grid=(N // R,),
        in_specs=[_ROW_SPEC, _ROW_SPEC, _ROW_SPEC, _MAT_SPEC, _MAT_SPEC, _VEC_SPEC],
        out_specs=[_ROW_SPEC, _ROW_SPEC],
        out_shape=[
            jax.ShapeDtypeStruct((N, 128), jnp.float32),
            jax.ShapeDtypeStruct((N, 128), jnp.float32),
        ],
        name="sage_tc1",
    )(sp0, sp1, x, wa, wb, b)


def _tc23(sp, x, invd, wa, wb, b, relu):
    return pl.pallas_call(
        functools.partial(_tc23_body, relu),
        grid=(N // R,),
        in_specs=[_ROW_SPEC, _ROW_SPEC, _ROW_SPEC, _MAT_SPEC, _MAT_SPEC, _VEC_SPEC],
        out_specs=_ROW_SPEC,
        out_shape=jax.ShapeDtypeStruct((N, 128), jnp.float32),
        name="sage_tc23",
    )(sp, x, invd, wa, wb, b)


def kernel(h, edge_index, W1, b1, W2, b2, W3, b3):
    f32 = jnp.float32
    src_r = edge_index[0].reshape(ER, CH)
    dst_r = edge_index[1].reshape(ER, CH)

    ones = jnp.ones((ZR, 128), f32)
    z = jnp.zeros((ZR, 128), f32)

    sp0, sp1 = _agg1(h, ones, src_r, dst_r, z)
    x1, invd = _tc1(sp0, sp1, h, W1[:128], W1[128:], b1.reshape(1, 128))
    s2 = _agg2(x1, src_r, dst_r, z)
    x2 = _tc23(s2, x1, invd, W2[:128], W2[128:], b2.reshape(1, 128), True)
    s3 = _agg2(x2, src_r, dst_r, z)
    x3 = _tc23(s3, x2, invd, W3[:128], W3[128:], b3.reshape(1, 128), False)
    return x3
